# baseline (speedup 1.0000x reference)
import jax
import jax.numpy as jnp
from jax import lax
from jax.experimental import pallas as pl
from jax.experimental.pallas import tpu as pltpu

_CHUNK = 4096
_NBUF = 3


def _router_body(x_hbm, w_ref, probs_hbm, logits_hbm,
                 xbuf, pbuf, lbuf, in_sem, p_sem, l_sem):
    n_tokens = x_hbm.shape[0]
    n_chunks = n_tokens // _CHUNK
    w = w_ref[...]

    def in_copy(chunk, buf):
        return pltpu.make_async_copy(
            x_hbm.at[pl.ds(chunk * _CHUNK, _CHUNK), :],
            xbuf.at[buf],
            in_sem.at[buf],
        )

    for b in range(_NBUF):
        in_copy(b, b).start()

    def step(i, carry):
        buf = lax.rem(i, _NBUF)
        in_copy(i, buf).wait()

        @pl.when(i >= 1)
        def _():
            pltpu.make_async_copy(
                pbuf, probs_hbm.at[pl.ds(0, _CHUNK), :], p_sem
            ).wait()
            pltpu.make_async_copy(
                lbuf, logits_hbm.at[pl.ds(0, _CHUNK), :], l_sem
            ).wait()

        x = xbuf[buf]
        logits = jnp.dot(x, w, preferred_element_type=jnp.float32)
        m = jnp.max(logits, axis=-1, keepdims=True)
        e = jnp.exp(logits - m)
        probs = e / jnp.sum(e, axis=-1, keepdims=True)
        pbuf[...] = probs
        lbuf[...] = logits

        pltpu.make_async_copy(
            pbuf, probs_hbm.at[pl.ds(i * _CHUNK, _CHUNK), :], p_sem
        ).start()
        pltpu.make_async_copy(
            lbuf, logits_hbm.at[pl.ds(i * _CHUNK, _CHUNK), :], l_sem
        ).start()

        @pl.when(i + _NBUF < n_chunks)
        def _():
            in_copy(i + _NBUF, buf).start()

        return carry

    lax.fori_loop(0, n_chunks, step, 0)

    pltpu.make_async_copy(
        pbuf, probs_hbm.at[pl.ds(0, _CHUNK), :], p_sem
    ).wait()
    pltpu.make_async_copy(
        lbuf, logits_hbm.at[pl.ds(0, _CHUNK), :], l_sem
    ).wait()


def kernel(inputs, num_experts, w):
    n_tokens, d = inputs.shape
    n_exp = w.shape[1]
    probs, logits = pl.pallas_call(
        _router_body,
        in_specs=[
            pl.BlockSpec(memory_space=pl.ANY),
            pl.BlockSpec(memory_space=pltpu.VMEM),
        ],
        out_specs=[
            pl.BlockSpec(memory_space=pl.ANY),
            pl.BlockSpec(memory_space=pl.ANY),
        ],
        out_shape=[
            jax.ShapeDtypeStruct((n_tokens, n_exp), jnp.float32),
            jax.ShapeDtypeStruct((n_tokens, n_exp), jnp.float32),
        ],
        scratch_shapes=[
            pltpu.VMEM((_NBUF, _CHUNK, d), jnp.float32),
            pltpu.VMEM((_CHUNK, n_exp), jnp.float32),
            pltpu.VMEM((_CHUNK, n_exp), jnp.float32),
            pltpu.SemaphoreType.DMA((_NBUF,)),
            pltpu.SemaphoreType.DMA,
            pltpu.SemaphoreType.DMA,
        ],
    )(inputs, w)
    return (probs, logits, 0)
